# baseline (speedup 1.0000x reference)
import jax
import jax.numpy as jnp
from jax import lax
from jax.experimental import pallas as pl
from jax.experimental.pallas import tpu as pltpu
from jax.experimental.pallas import tpu_sc as plsc

_CH = 256
_SUB = 128
_NSUB = _CH // _SUB
_NT = 16
_NC = 2


def _ceil_to(x, m):
    return -(-x // m) * m


def _write_stripes(acc_sp, out_h, row0, s, opt8, lastn, sem):
    kcom = lastn // _SUB
    ocs = []
    for k in range(kcom):
        ocs.append(pltpu.async_copy(
            acc_sp.at[pl.ds(s * opt8 + k * _SUB, _SUB)],
            out_h.at[pl.ds(row0 + s * opt8 + k * _SUB, _SUB)], sem))
    t_full = opt8 - kcom * _SUB
    t_last = lastn - kcom * _SUB
    if t_full == t_last:
        if t_full:
            ocs.append(pltpu.async_copy(
                acc_sp.at[pl.ds(s * opt8 + kcom * _SUB, t_full)],
                out_h.at[pl.ds(row0 + s * opt8 + kcom * _SUB, t_full)], sem))
    else:
        @pl.when(s < _NT - 1)
        def _():
            if t_full:
                pltpu.async_copy(
                    acc_sp.at[pl.ds(s * opt8 + kcom * _SUB, t_full)],
                    out_h.at[pl.ds(row0 + s * opt8 + kcom * _SUB, t_full)],
                    sem).wait()

        @pl.when(s == _NT - 1)
        def _():
            if t_last:
                pltpu.async_copy(
                    acc_sp.at[pl.ds(s * opt8 + kcom * _SUB, t_last)],
                    out_h.at[pl.ds(row0 + s * opt8 + kcom * _SUB, t_last)],
                    sem).wait()
    for cp in ocs:
        cp.wait()


def _build_k0(N, R, E_pad):
    LPT = E_pad // _NT
    NCH = LPT // _CH
    LPW = E_pad // (_NC * _NT)
    NCHW = LPW // _CH
    CWPT = _ceil_to(-(-(N * R + 1) // _NT), 128)
    CNTW = _NT * CWPT

    def body(ed_h, norm_h,
             cnt_sp, ed_a, ed_b, pair_a, pair_b, cntv, normv, onesv, zb1,
             sem_a, sem_b):
        c = lax.axis_index("c")
        s = lax.axis_index("s")
        on = jnp.ones((16,), jnp.float32)
        zf = jnp.zeros((16,), jnp.float32)
        eds = (ed_a, ed_b)
        pairs = (pair_a, pair_b)

        for g in range(_SUB // 16):
            onesv[pl.ds(g * 16, 16)] = on

        def zrow1(k, carry):
            zb1[pl.ds(k * 16, 16)] = zf
            return carry
        lax.fori_loop(0, 4096 // 16, zrow1, 0)

        wz = s * CWPT
        wfull = CWPT // 4096
        zcs = []
        for k in range(wfull):
            zcs.append(pltpu.async_copy(
                zb1, cnt_sp.at[pl.ds(wz + k * 4096, 4096)], sem_a))
        wrem = CWPT - wfull * 4096
        if wrem:
            zcs.append(pltpu.async_copy(
                zb1.at[pl.ds(0, wrem)],
                cnt_sp.at[pl.ds(wz + wfull * 4096, wrem)], sem_a))
        for cp in zcs:
            cp.wait()
        plsc.subcore_barrier()

        blk0 = s * (LPT // _SUB)

        def cpair(q):
            for j in range(_NSUB):
                for g in range(_SUB // 16):
                    sl = pl.ds(g * 16, 16)
                    pairs[q][j, sl] = (eds[q][j, 1, sl] * R
                                       + eds[q][j, 2, sl])

        pltpu.sync_copy(ed_h.at[pl.ds(blk0, _NSUB)], ed_a)

        def count_chunk(i, carry):
            for q in range(2):
                ii = 2 * i + q
                nblk = blk0 + jnp.minimum(ii + 1, NCH - 1) * _NSUB
                nxt = pltpu.async_copy(ed_h.at[pl.ds(nblk, _NSUB)],
                                       eds[1 - q], sem_a)
                cpair(q)
                scs = [pltpu.async_copy(onesv, cnt_sp.at[pairs[q].at[j]],
                                        sem_b, add=True)
                       for j in range(_NSUB)]
                nxt.wait()
                for cp in scs:
                    cp.wait()
            return carry
        lax.fori_loop(0, NCH // 2, count_chunk, 0)
        plsc.subcore_barrier()

        wblk0 = (c * _NT + s) * (LPW // _SUB)
        pltpu.sync_copy(ed_h.at[pl.ds(wblk0, _NSUB)], ed_a)

        def norm_chunk(i, carry):
            for q in range(2):
                ii = 2 * i + q
                nblk = wblk0 + jnp.minimum(ii + 1, NCHW - 1) * _NSUB
                nxt = pltpu.async_copy(ed_h.at[pl.ds(nblk, _NSUB)],
                                       eds[1 - q], sem_a)
                cpair(q)
                gcs = [pltpu.async_copy(cnt_sp.at[pairs[q].at[j]],
                                        cntv.at[pl.ds(j * _SUB, _SUB)],
                                        sem_b)
                       for j in range(_NSUB)]
                for cp in gcs:
                    cp.wait()
                for g in range(_CH // 16):
                    sl = pl.ds(g * 16, 16)
                    normv[sl] = 1.0 / cntv[sl]
                base = (wblk0 + ii * _NSUB) * _SUB
                pltpu.sync_copy(normv, norm_h.at[pl.ds(base, _CH)])
                nxt.wait()
            return carry
        lax.fori_loop(0, NCHW // 2, norm_chunk, 0)


    scratch = [
        pltpu.VMEM_SHARED((CNTW,), jnp.float32),
        pltpu.VMEM((_NSUB, 3, _SUB), jnp.int32),
        pltpu.VMEM((_NSUB, 3, _SUB), jnp.int32),
        pltpu.VMEM((_NSUB, _SUB), jnp.int32),
        pltpu.VMEM((_NSUB, _SUB), jnp.int32),
        pltpu.VMEM((_CH,), jnp.float32),
        pltpu.VMEM((_CH,), jnp.float32),
        pltpu.VMEM((_SUB,), jnp.float32),
        pltpu.VMEM((4096,), jnp.float32),
        pltpu.SemaphoreType.DMA,
        pltpu.SemaphoreType.DMA,
    ]
    return body, scratch


def _build_layer(N, R, W, E_pad, layer1):
    LPT = E_pad // _NT if layer1 else E_pad // (_NC * _NT)
    NCH = LPT // _CH
    OPT8 = _ceil_to(-(-N // _NT), 8)
    NROW = _NT * OPT8
    LASTN = N - (_NT - 1) * OPT8

    def body(ed_h, norm_h, tab_h, out_h,
             acc_sp, ed_a, ed_b, idx_a, idx_b, nrm_a, nrm_b,
             rows_a, rows_b, rows2,
             sed_a, sed_b, sg_a, sg_b, ss):
        c = lax.axis_index("c")
        s = lax.axis_index("s")
        i16 = lax.broadcasted_iota(jnp.int32, (16,), 0)
        zf = jnp.zeros((16,), jnp.float32)
        eds = (ed_a, ed_b)
        idxs = (idx_a, idx_b)
        nrms = (nrm_a, nrm_b)
        rowss = (rows_a, rows_b)
        seds = (sed_a, sed_b)
        sgs = (sg_a, sg_b)

        def zrow(r, carry):
            for qq in range(W // 16):
                rows2[r, pl.ds(qq * 16, 16)] = zf
            return carry
        lax.fori_loop(0, _CH, zrow, 0)
        rz = s * OPT8
        nfull = OPT8 // _CH
        zcs = [pltpu.async_copy(rows2, acc_sp.at[pl.ds(rz + k * _CH, _CH)],
                                sg_a)
               for k in range(nfull)]
        rem = OPT8 - nfull * _CH
        if rem:
            zcs.append(pltpu.async_copy(
                rows2.at[pl.ds(0, rem)],
                acc_sp.at[pl.ds(rz + nfull * _CH, rem)], sg_a))
        for cp in zcs:
            cp.wait()
        plsc.subcore_barrier()

        if layer1:
            blk0 = s * (LPT // _SUB)
        else:
            blk0 = (c * _NT + s) * (LPT // _SUB)

        def blk(i):
            return blk0 + i * _NSUB

        def ebase(i):
            return blk(i) * _SUB

        def load_ed(i, q, sem):
            return [pltpu.async_copy(ed_h.at[pl.ds(blk(i), _NSUB)], eds[q],
                                     sem),
                    pltpu.async_copy(norm_h.at[pl.ds(ebase(i), _CH)],
                                     nrms[q], sem)]

        def wait_ed(q):
            pltpu.make_async_copy(ed_h.at[pl.ds(0, _NSUB)], eds[q],
                                  seds[q]).wait()
            pltpu.make_async_copy(norm_h.at[pl.ds(0, _CH)], nrms[q],
                                  seds[q]).wait()

        def comp_idx(q):
            for j in range(_NSUB):
                for g in range(_SUB // 16):
                    sl = pl.ds(g * 16, 16)
                    sv = eds[q][j, 0, sl]
                    tv = eds[q][j, 2, sl]
                    if layer1:
                        idxs[q][j, sl] = (tv * N + sv) * 2 + c
                    else:
                        idxs[q][j, sl] = sv * R + tv

        def fire_gather(q):
            return [pltpu.async_copy(tab_h.at[idxs[q].at[j]],
                                     rowss[q].at[pl.ds(j * _SUB, _SUB)],
                                     sgs[q])
                    for j in range(_NSUB)]

        def wait_gather(q):
            for j in range(_NSUB):
                pltpu.make_async_copy(tab_h.at[idxs[q].at[j]],
                                      rowss[q].at[pl.ds(j * _SUB, _SUB)],
                                      sgs[q]).wait()

        def scale(q):
            def sgroup(k, carry):
                fo = k * 16
                rid = i16 + fo
                nv = nrms[q][pl.ds(fo, 16)]
                cids = [jnp.full((16,), col, jnp.int32) for col in range(W)]
                vals = [plsc.load_gather(rowss[q], [rid, cids[col]]) * nv
                        for col in range(W)]
                for col in range(W):
                    plsc.store_scatter(rows2, [rid, cids[col]], vals[col])
                return carry
            lax.fori_loop(0, _CH // 16, sgroup, 0)

        def fire_scatter(q):
            return [pltpu.async_copy(rows2.at[pl.ds(j * _SUB, _SUB)],
                                     acc_sp.at[eds[q].at[j, 1]], ss,
                                     add=True)
                    for j in range(_NSUB)]

        pltpu.sync_copy(ed_h.at[pl.ds(blk(0), _NSUB)], ed_a)
        pltpu.sync_copy(norm_h.at[pl.ds(ebase(0), _CH)], nrm_a)
        comp_idx(0)
        fire_gather(0)
        load_ed(jnp.minimum(1, NCH - 1), 1, sed_b)

        def step(i, p):
            q, r = p, 1 - p
            wait_ed(r)
            comp_idx(r)
            wait_gather(q)
            fire_gather(r)
            scale(q)
            scs = fire_scatter(q)
            for cp in scs:
                cp.wait()
            load_ed(jnp.minimum(i + 2, NCH - 1), q, seds[q])

        def pair_step(t, carry):
            step(2 * t, 0)
            step(2 * t + 1, 1)
            return carry
        lax.fori_loop(0, NCH // 2, pair_step, 0)

        wait_gather(0)
        wait_ed(1)
        plsc.subcore_barrier()

        _write_stripes(acc_sp, out_h, c * N, s, OPT8, LASTN, ss)

    scratch = [
        pltpu.VMEM_SHARED((NROW, W), jnp.float32),
        pltpu.VMEM((_NSUB, 3, _SUB), jnp.int32),
        pltpu.VMEM((_NSUB, 3, _SUB), jnp.int32),
        pltpu.VMEM((_NSUB, _SUB), jnp.int32),
        pltpu.VMEM((_NSUB, _SUB), jnp.int32),
        pltpu.VMEM((_CH,), jnp.float32),
        pltpu.VMEM((_CH,), jnp.float32),
        pltpu.VMEM((_CH, W), jnp.float32),
        pltpu.VMEM((_CH, W), jnp.float32),
        pltpu.VMEM((_CH, W), jnp.float32),
        pltpu.SemaphoreType.DMA,
        pltpu.SemaphoreType.DMA,
        pltpu.SemaphoreType.DMA,
        pltpu.SemaphoreType.DMA,
        pltpu.SemaphoreType.DMA,
    ]
    return body, scratch


def _dense_body(h0, h1, r1, b1r, w2c, r2, xr_o, hr_o):
    h = jnp.concatenate([h0[...], h1[...]], axis=1)
    h = jnp.maximum(h + r1[...] + b1r[...], 0.0)
    xr_o[...] = jnp.dot(h, w2c[...], preferred_element_type=jnp.float32)
    hr_o[...] = jnp.dot(h, r2[...], preferred_element_type=jnp.float32)


def _finish_body(p0, p1, hr, b2r, o):
    x = p0[...] + p1[...] + hr[...] + b2r[...]
    o[...] = 1.0 / (1.0 + jnp.exp(-x))


def kernel(edge_index, edge_type, W1, root1, b1, W2, root2, b2):
    R, N, H = W1.shape
    L = W2.shape[2]
    E = edge_index.shape[1]
    Hh = H // 2

    E_pad = _ceil_to(E, 2 * _NC * _NT * _CH)
    pad = E_pad - E

    src = edge_index[0].astype(jnp.int32)
    dst = edge_index[1].astype(jnp.int32)
    typ = edge_type.astype(jnp.int32)
    src_p = jnp.concatenate([src, jnp.zeros((pad,), jnp.int32)])
    dst_p = jnp.concatenate([dst, jnp.full((pad,), N, jnp.int32)])
    typ_p = jnp.concatenate([typ, jnp.zeros((pad,), jnp.int32)])
    epak = jnp.stack([src_p.reshape(-1, _SUB), dst_p.reshape(-1, _SUB),
                      typ_p.reshape(-1, _SUB)], axis=1)

    w1v = W1.reshape(R * N * 2, Hh)

    mesh = plsc.VectorSubcoreMesh(core_axis_name="c", subcore_axis_name="s")
    sc_params = pltpu.CompilerParams(needs_layout_passes=False,
                                     use_tc_tiling_on_sc=False)

    k0_body, k0_scratch = _build_k0(N, R, E_pad)
    norm_e = pl.kernel(
        k0_body,
        out_type=jax.ShapeDtypeStruct((E_pad,), jnp.float32),
        mesh=mesh,
        compiler_params=sc_params,
        scratch_types=k0_scratch,
    )(epak)

    k1_body, k1_scratch = _build_layer(N, R, Hh, E_pad, True)
    hpre = pl.kernel(
        k1_body,
        out_type=jax.ShapeDtypeStruct((_NC * N, Hh), jnp.float32),
        mesh=mesh,
        compiler_params=sc_params,
        scratch_types=k1_scratch,
    )(epak, norm_e, w1v)

    BN = 1000
    NB = N // BN
    w2c = jnp.transpose(W2, (1, 0, 2)).reshape(H, R * L)
    xr2d, hroot = pl.pallas_call(
        _dense_body,
        grid=(NB,),
        in_specs=[
            pl.BlockSpec((BN, Hh), lambda i: (i, 0)),
            pl.BlockSpec((BN, Hh), lambda i, _nb=NB: (i + _nb, 0)),
            pl.BlockSpec((BN, H), lambda i: (i, 0)),
            pl.BlockSpec((1, H), lambda i: (0, 0)),
            pl.BlockSpec((H, R * L), lambda i: (0, 0)),
            pl.BlockSpec((H, L), lambda i: (0, 0)),
        ],
        out_specs=[pl.BlockSpec((BN, R * L), lambda i: (i, 0)),
                   pl.BlockSpec((BN, L), lambda i: (i, 0))],
        out_shape=[jax.ShapeDtypeStruct((N, R * L), jnp.float32),
                   jax.ShapeDtypeStruct((N, L), jnp.float32)],
    )(hpre, hpre, root1, b1.reshape(1, H), w2c, root2)

    xrv = xr2d.reshape(N * R, L)

    k3_body, k3_scratch = _build_layer(N, R, L, E_pad, False)
    opart = pl.kernel(
        k3_body,
        out_type=jax.ShapeDtypeStruct((_NC * N, L), jnp.float32),
        mesh=mesh,
        compiler_params=sc_params,
        scratch_types=k3_scratch,
    )(epak, norm_e, xrv)

    out = pl.pallas_call(
        _finish_body,
        grid=(NB,),
        in_specs=[
            pl.BlockSpec((BN, L), lambda i: (i, 0)),
            pl.BlockSpec((BN, L), lambda i, _nb=NB: (i + _nb, 0)),
            pl.BlockSpec((BN, L), lambda i: (i, 0)),
            pl.BlockSpec((1, L), lambda i: (0, 0)),
        ],
        out_specs=pl.BlockSpec((BN, L), lambda i: (i, 0)),
        out_shape=jax.ShapeDtypeStruct((N, L), jnp.float32),
    )(opart, opart, hroot, b2.reshape(1, L))
    return out


# speedup vs baseline: 2.2437x; 2.2437x over previous
import jax
import jax.numpy as jnp
from jax import lax
from jax.experimental import pallas as pl
from jax.experimental.pallas import tpu as pltpu
from jax.experimental.pallas import tpu_sc as plsc

_CH = 256
_SUB = 128
_NSUB = _CH // _SUB
_NT = 16
_NC = 2


def _ceil_to(x, m):
    return -(-x // m) * m


def _write_stripes(acc_sp, out_h, row0, s, opt8, lastn, sem):
    kcom = lastn // _SUB
    ocs = []
    for k in range(kcom):
        ocs.append(pltpu.async_copy(
            acc_sp.at[pl.ds(s * opt8 + k * _SUB, _SUB)],
            out_h.at[pl.ds(row0 + s * opt8 + k * _SUB, _SUB)], sem))
    t_full = opt8 - kcom * _SUB
    t_last = lastn - kcom * _SUB
    if t_full == t_last:
        if t_full:
            ocs.append(pltpu.async_copy(
                acc_sp.at[pl.ds(s * opt8 + kcom * _SUB, t_full)],
                out_h.at[pl.ds(row0 + s * opt8 + kcom * _SUB, t_full)], sem))
    else:
        @pl.when(s < _NT - 1)
        def _():
            if t_full:
                pltpu.async_copy(
                    acc_sp.at[pl.ds(s * opt8 + kcom * _SUB, t_full)],
                    out_h.at[pl.ds(row0 + s * opt8 + kcom * _SUB, t_full)],
                    sem).wait()

        @pl.when(s == _NT - 1)
        def _():
            if t_last:
                pltpu.async_copy(
                    acc_sp.at[pl.ds(s * opt8 + kcom * _SUB, t_last)],
                    out_h.at[pl.ds(row0 + s * opt8 + kcom * _SUB, t_last)],
                    sem).wait()
    for cp in ocs:
        cp.wait()


def _build_k0(N, R, E_pad):
    LPT = E_pad // _NT
    NCH = LPT // _CH
    LPW = E_pad // (_NC * _NT)
    NCHW = LPW // _CH
    CWPT = _ceil_to(-(-(N * R + 1) // _NT), 128)
    CNTW = _NT * CWPT

    def body(ed_h, norm_h,
             cnt_sp, ed_a, ed_b, pair_a, pair_b, cntv, normv, onesv, zb1,
             sem_a, sem_b):
        c = lax.axis_index("c")
        s = lax.axis_index("s")
        on = jnp.ones((16,), jnp.float32)
        zf = jnp.zeros((16,), jnp.float32)
        eds = (ed_a, ed_b)
        pairs = (pair_a, pair_b)

        for g in range(_SUB // 16):
            onesv[pl.ds(g * 16, 16)] = on

        def zrow1(k, carry):
            zb1[pl.ds(k * 16, 16)] = zf
            return carry
        lax.fori_loop(0, 4096 // 16, zrow1, 0)

        wz = s * CWPT
        wfull = CWPT // 4096
        zcs = []
        for k in range(wfull):
            zcs.append(pltpu.async_copy(
                zb1, cnt_sp.at[pl.ds(wz + k * 4096, 4096)], sem_a))
        wrem = CWPT - wfull * 4096
        if wrem:
            zcs.append(pltpu.async_copy(
                zb1.at[pl.ds(0, wrem)],
                cnt_sp.at[pl.ds(wz + wfull * 4096, wrem)], sem_a))
        for cp in zcs:
            cp.wait()
        plsc.subcore_barrier()

        blk0 = s * (LPT // _SUB)

        def cpair(q):
            for j in range(_NSUB):
                for g in range(_SUB // 16):
                    sl = pl.ds(g * 16, 16)
                    pairs[q][j, sl] = (eds[q][j, 1, sl] * R
                                       + eds[q][j, 2, sl])

        pltpu.sync_copy(ed_h.at[pl.ds(blk0, _NSUB)], ed_a)

        def count_chunk(i, carry):
            for q in range(2):
                ii = 2 * i + q
                nblk = blk0 + jnp.minimum(ii + 1, NCH - 1) * _NSUB
                nxt = pltpu.async_copy(ed_h.at[pl.ds(nblk, _NSUB)],
                                       eds[1 - q], sem_a)
                cpair(q)
                scs = [pltpu.async_copy(onesv, cnt_sp.at[pairs[q].at[j]],
                                        sem_b, add=True)
                       for j in range(_NSUB)]
                nxt.wait()
                for cp in scs:
                    cp.wait()
            return carry
        lax.fori_loop(0, NCH // 2, count_chunk, 0)
        plsc.subcore_barrier()

        wblk0 = (c * _NT + s) * (LPW // _SUB)
        pltpu.sync_copy(ed_h.at[pl.ds(wblk0, _NSUB)], ed_a)

        def norm_chunk(i, carry):
            for q in range(2):
                ii = 2 * i + q
                nblk = wblk0 + jnp.minimum(ii + 1, NCHW - 1) * _NSUB
                nxt = pltpu.async_copy(ed_h.at[pl.ds(nblk, _NSUB)],
                                       eds[1 - q], sem_a)
                cpair(q)
                gcs = [pltpu.async_copy(cnt_sp.at[pairs[q].at[j]],
                                        cntv.at[pl.ds(j * _SUB, _SUB)],
                                        sem_b)
                       for j in range(_NSUB)]
                for cp in gcs:
                    cp.wait()
                for g in range(_CH // 16):
                    sl = pl.ds(g * 16, 16)
                    normv[sl] = 1.0 / cntv[sl]
                base = (wblk0 + ii * _NSUB) * _SUB
                pltpu.sync_copy(normv, norm_h.at[pl.ds(base, _CH)])
                nxt.wait()
            return carry
        lax.fori_loop(0, NCHW // 2, norm_chunk, 0)


    scratch = [
        pltpu.VMEM_SHARED((CNTW,), jnp.float32),
        pltpu.VMEM((_NSUB, 3, _SUB), jnp.int32),
        pltpu.VMEM((_NSUB, 3, _SUB), jnp.int32),
        pltpu.VMEM((_NSUB, _SUB), jnp.int32),
        pltpu.VMEM((_NSUB, _SUB), jnp.int32),
        pltpu.VMEM((_CH,), jnp.float32),
        pltpu.VMEM((_CH,), jnp.float32),
        pltpu.VMEM((_SUB,), jnp.float32),
        pltpu.VMEM((4096,), jnp.float32),
        pltpu.SemaphoreType.DMA,
        pltpu.SemaphoreType.DMA,
    ]
    return body, scratch


def _build_layer(N, R, W, E_pad, layer1):
    LPT = E_pad // _NT if layer1 else E_pad // (_NC * _NT)
    NCH = LPT // _CH
    OPT8 = _ceil_to(-(-N // _NT), 8)
    NROW = _NT * OPT8
    LASTN = N - (_NT - 1) * OPT8

    def body(ed_h, norm_h, tab_h, out_h,
             acc_sp, ed_a, ed_b, idx_a, idx_b, nrm_a, nrm_b,
             rows_a, rows_b, rows2,
             sed_a, sed_b, sg_a, sg_b, ss):
        c = lax.axis_index("c")
        s = lax.axis_index("s")
        i16 = lax.broadcasted_iota(jnp.int32, (16,), 0)
        zf = jnp.zeros((16,), jnp.float32)
        eds = (ed_a, ed_b)
        idxs = (idx_a, idx_b)
        nrms = (nrm_a, nrm_b)
        rowss = (rows_a, rows_b)
        seds = (sed_a, sed_b)
        sgs = (sg_a, sg_b)

        def zrow(r, carry):
            for qq in range(W // 16):
                rows2[r, pl.ds(qq * 16, 16)] = zf
            return carry
        lax.fori_loop(0, _CH, zrow, 0)
        rz = s * OPT8
        nfull = OPT8 // _CH
        zcs = [pltpu.async_copy(rows2, acc_sp.at[pl.ds(rz + k * _CH, _CH)],
                                sg_a)
               for k in range(nfull)]
        rem = OPT8 - nfull * _CH
        if rem:
            zcs.append(pltpu.async_copy(
                rows2.at[pl.ds(0, rem)],
                acc_sp.at[pl.ds(rz + nfull * _CH, rem)], sg_a))
        for cp in zcs:
            cp.wait()
        plsc.subcore_barrier()

        if layer1:
            blk0 = s * (LPT // _SUB)
        else:
            blk0 = (c * _NT + s) * (LPT // _SUB)

        def blk(i):
            return blk0 + i * _NSUB

        def ebase(i):
            return blk(i) * _SUB

        def load_ed(i, q, sem):
            return [pltpu.async_copy(ed_h.at[pl.ds(blk(i), _NSUB)], eds[q],
                                     sem),
                    pltpu.async_copy(norm_h.at[pl.ds(ebase(i), _CH)],
                                     nrms[q], sem)]

        def wait_ed(q):
            pltpu.make_async_copy(ed_h.at[pl.ds(0, _NSUB)], eds[q],
                                  seds[q]).wait()
            pltpu.make_async_copy(norm_h.at[pl.ds(0, _CH)], nrms[q],
                                  seds[q]).wait()

        def comp_idx(q):
            for j in range(_NSUB):
                for g in range(_SUB // 16):
                    sl = pl.ds(g * 16, 16)
                    sv = eds[q][j, 0, sl]
                    tv = eds[q][j, 2, sl]
                    if layer1:
                        idxs[q][j, sl] = (tv * N + sv) * 2 + c
                    else:
                        idxs[q][j, sl] = sv * R + tv

        def fire_gather(q):
            return [pltpu.async_copy(tab_h.at[idxs[q].at[j]],
                                     rowss[q].at[pl.ds(j * _SUB, _SUB)],
                                     sgs[q])
                    for j in range(_NSUB)]

        def wait_gather(q):
            for j in range(_NSUB):
                pltpu.make_async_copy(tab_h.at[idxs[q].at[j]],
                                      rowss[q].at[pl.ds(j * _SUB, _SUB)],
                                      sgs[q]).wait()

        def scale(q):
            def sgroup(k, carry):
                fo = k * 16
                nv = nrms[q][pl.ds(fo, 16)]
                for h in range(2):
                    vals = []
                    for g2 in range(8):
                        e = fo + h * 8 + g2
                        s0 = nv[h * 8 + g2]
                        vals.append([rowss[q][e, pl.ds(c16 * 16, 16)] * s0
                                     for c16 in range(W // 16)])
                    for g2 in range(8):
                        e = fo + h * 8 + g2
                        for c16 in range(W // 16):
                            rows2[e, pl.ds(c16 * 16, 16)] = vals[g2][c16]
                return carry
            lax.fori_loop(0, _CH // 16, sgroup, 0)

        def fire_scatter(q):
            return [pltpu.async_copy(rows2.at[pl.ds(j * _SUB, _SUB)],
                                     acc_sp.at[eds[q].at[j, 1]], ss,
                                     add=True)
                    for j in range(_NSUB)]

        pltpu.sync_copy(ed_h.at[pl.ds(blk(0), _NSUB)], ed_a)
        pltpu.sync_copy(norm_h.at[pl.ds(ebase(0), _CH)], nrm_a)
        comp_idx(0)
        fire_gather(0)
        load_ed(jnp.minimum(1, NCH - 1), 1, sed_b)

        def step(i, p):
            q, r = p, 1 - p
            wait_ed(r)
            comp_idx(r)
            wait_gather(q)
            fire_gather(r)
            scale(q)
            scs = fire_scatter(q)
            for cp in scs:
                cp.wait()
            load_ed(jnp.minimum(i + 2, NCH - 1), q, seds[q])

        def pair_step(t, carry):
            step(2 * t, 0)
            step(2 * t + 1, 1)
            return carry
        lax.fori_loop(0, NCH // 2, pair_step, 0)

        wait_gather(0)
        wait_ed(1)
        plsc.subcore_barrier()

        _write_stripes(acc_sp, out_h, c * N, s, OPT8, LASTN, ss)

    scratch = [
        pltpu.VMEM_SHARED((NROW, W), jnp.float32),
        pltpu.VMEM((_NSUB, 3, _SUB), jnp.int32),
        pltpu.VMEM((_NSUB, 3, _SUB), jnp.int32),
        pltpu.VMEM((_NSUB, _SUB), jnp.int32),
        pltpu.VMEM((_NSUB, _SUB), jnp.int32),
        pltpu.VMEM((_CH,), jnp.float32),
        pltpu.VMEM((_CH,), jnp.float32),
        pltpu.VMEM((_CH, W), jnp.float32),
        pltpu.VMEM((_CH, W), jnp.float32),
        pltpu.VMEM((_CH, W), jnp.float32),
        pltpu.SemaphoreType.DMA,
        pltpu.SemaphoreType.DMA,
        pltpu.SemaphoreType.DMA,
        pltpu.SemaphoreType.DMA,
        pltpu.SemaphoreType.DMA,
    ]
    return body, scratch


def _dense_body(h0, h1, r1, b1r, w2c, r2, xr_o, hr_o):
    h = jnp.concatenate([h0[...], h1[...]], axis=1)
    h = jnp.maximum(h + r1[...] + b1r[...], 0.0)
    xr_o[...] = jnp.dot(h, w2c[...], preferred_element_type=jnp.float32)
    hr_o[...] = jnp.dot(h, r2[...], preferred_element_type=jnp.float32)


def _finish_body(p0, p1, hr, b2r, o):
    x = p0[...] + p1[...] + hr[...] + b2r[...]
    o[...] = 1.0 / (1.0 + jnp.exp(-x))


def kernel(edge_index, edge_type, W1, root1, b1, W2, root2, b2):
    R, N, H = W1.shape
    L = W2.shape[2]
    E = edge_index.shape[1]
    Hh = H // 2

    E_pad = _ceil_to(E, 2 * _NC * _NT * _CH)
    pad = E_pad - E

    src = edge_index[0].astype(jnp.int32)
    dst = edge_index[1].astype(jnp.int32)
    typ = edge_type.astype(jnp.int32)
    src_p = jnp.concatenate([src, jnp.zeros((pad,), jnp.int32)])
    dst_p = jnp.concatenate([dst, jnp.full((pad,), N, jnp.int32)])
    typ_p = jnp.concatenate([typ, jnp.zeros((pad,), jnp.int32)])
    epak = jnp.stack([src_p.reshape(-1, _SUB), dst_p.reshape(-1, _SUB),
                      typ_p.reshape(-1, _SUB)], axis=1)

    w1v = W1.reshape(R * N * 2, Hh)

    mesh = plsc.VectorSubcoreMesh(core_axis_name="c", subcore_axis_name="s")
    sc_params = pltpu.CompilerParams(needs_layout_passes=False,
                                     use_tc_tiling_on_sc=False)

    k0_body, k0_scratch = _build_k0(N, R, E_pad)
    norm_e = pl.kernel(
        k0_body,
        out_type=jax.ShapeDtypeStruct((E_pad,), jnp.float32),
        mesh=mesh,
        compiler_params=sc_params,
        scratch_types=k0_scratch,
    )(epak)

    k1_body, k1_scratch = _build_layer(N, R, Hh, E_pad, True)
    hpre = pl.kernel(
        k1_body,
        out_type=jax.ShapeDtypeStruct((_NC * N, Hh), jnp.float32),
        mesh=mesh,
        compiler_params=sc_params,
        scratch_types=k1_scratch,
    )(epak, norm_e, w1v)

    BN = 1000
    NB = N // BN
    w2c = jnp.transpose(W2, (1, 0, 2)).reshape(H, R * L)
    xr2d, hroot = pl.pallas_call(
        _dense_body,
        grid=(NB,),
        in_specs=[
            pl.BlockSpec((BN, Hh), lambda i: (i, 0)),
            pl.BlockSpec((BN, Hh), lambda i, _nb=NB: (i + _nb, 0)),
            pl.BlockSpec((BN, H), lambda i: (i, 0)),
            pl.BlockSpec((1, H), lambda i: (0, 0)),
            pl.BlockSpec((H, R * L), lambda i: (0, 0)),
            pl.BlockSpec((H, L), lambda i: (0, 0)),
        ],
        out_specs=[pl.BlockSpec((BN, R * L), lambda i: (i, 0)),
                   pl.BlockSpec((BN, L), lambda i: (i, 0))],
        out_shape=[jax.ShapeDtypeStruct((N, R * L), jnp.float32),
                   jax.ShapeDtypeStruct((N, L), jnp.float32)],
    )(hpre, hpre, root1, b1.reshape(1, H), w2c, root2)

    xrv = xr2d.reshape(N * R, L)

    k3_body, k3_scratch = _build_layer(N, R, L, E_pad, False)
    opart = pl.kernel(
        k3_body,
        out_type=jax.ShapeDtypeStruct((_NC * N, L), jnp.float32),
        mesh=mesh,
        compiler_params=sc_params,
        scratch_types=k3_scratch,
    )(epak, norm_e, xrv)

    out = pl.pallas_call(
        _finish_body,
        grid=(NB,),
        in_specs=[
            pl.BlockSpec((BN, L), lambda i: (i, 0)),
            pl.BlockSpec((BN, L), lambda i, _nb=NB: (i + _nb, 0)),
            pl.BlockSpec((BN, L), lambda i: (i, 0)),
            pl.BlockSpec((1, L), lambda i: (0, 0)),
        ],
        out_specs=pl.BlockSpec((BN, L), lambda i: (i, 0)),
        out_shape=jax.ShapeDtypeStruct((N, L), jnp.float32),
    )(opart, opart, hroot, b2.reshape(1, L))
    return out


# speedup vs baseline: 2.2574x; 1.0061x over previous
import jax
import jax.numpy as jnp
from jax import lax
from jax.experimental import pallas as pl
from jax.experimental.pallas import tpu as pltpu
from jax.experimental.pallas import tpu_sc as plsc

_CH = 256
_SUB = 128
_NSUB = _CH // _SUB
_NT = 16
_NC = 2


def _ceil_to(x, m):
    return -(-x // m) * m


def _write_stripes(acc_sp, out_h, row0, s, opt8, lastn, sem):
    kcom = lastn // _SUB
    ocs = []
    for k in range(kcom):
        ocs.append(pltpu.async_copy(
            acc_sp.at[pl.ds(s * opt8 + k * _SUB, _SUB)],
            out_h.at[pl.ds(row0 + s * opt8 + k * _SUB, _SUB)], sem))
    t_full = opt8 - kcom * _SUB
    t_last = lastn - kcom * _SUB
    if t_full == t_last:
        if t_full:
            ocs.append(pltpu.async_copy(
                acc_sp.at[pl.ds(s * opt8 + kcom * _SUB, t_full)],
                out_h.at[pl.ds(row0 + s * opt8 + kcom * _SUB, t_full)], sem))
    else:
        @pl.when(s < _NT - 1)
        def _():
            if t_full:
                pltpu.async_copy(
                    acc_sp.at[pl.ds(s * opt8 + kcom * _SUB, t_full)],
                    out_h.at[pl.ds(row0 + s * opt8 + kcom * _SUB, t_full)],
                    sem).wait()

        @pl.when(s == _NT - 1)
        def _():
            if t_last:
                pltpu.async_copy(
                    acc_sp.at[pl.ds(s * opt8 + kcom * _SUB, t_last)],
                    out_h.at[pl.ds(row0 + s * opt8 + kcom * _SUB, t_last)],
                    sem).wait()
    for cp in ocs:
        cp.wait()


def _build_k0(N, R, E_pad):
    LPT = E_pad // _NT
    NCH = LPT // _CH
    LPW = E_pad // (_NC * _NT)
    NCHW = LPW // _CH
    CWPT = _ceil_to(-(-(N * R + 1) // _NT), 128)
    CNTW = _NT * CWPT

    def body(dst2_h, typ_h, norm_h,
             cnt_sp, dst_a, dst_b, typ_a, typ_b, pair_a, pair_b,
             cntv, normv, onesv, zb1, sem_a, sem_b):
        c = lax.axis_index("c")
        s = lax.axis_index("s")
        on = jnp.ones((16,), jnp.float32)
        zf = jnp.zeros((16,), jnp.float32)
        zi = jnp.zeros((16,), jnp.int32)
        dsts = (dst_a, dst_b)
        typs = (typ_a, typ_b)
        pairs = (pair_a, pair_b)

        for g in range(_SUB // 16):
            onesv[pl.ds(g * 16, 16)] = on
        for q in range(2):
            for j in range(_NSUB):
                for g in range(_SUB // 16):
                    pairs[q][j, pl.ds(g * 16, 16)] = zi

        def zrow1(k, carry):
            zb1[pl.ds(k * 16, 16)] = zf
            return carry
        lax.fori_loop(0, 4096 // 16, zrow1, 0)

        wz = s * CWPT
        wfull = CWPT // 4096
        zcs = []
        for k in range(wfull):
            zcs.append(pltpu.async_copy(
                zb1, cnt_sp.at[pl.ds(wz + k * 4096, 4096)], sem_a))
        wrem = CWPT - wfull * 4096
        if wrem:
            zcs.append(pltpu.async_copy(
                zb1.at[pl.ds(0, wrem)],
                cnt_sp.at[pl.ds(wz + wfull * 4096, wrem)], sem_a))
        for cp in zcs:
            cp.wait()
        plsc.subcore_barrier()

        def led(i, q, blk_base):
            b = blk_base + i * _NSUB
            return [pltpu.async_copy(dst2_h.at[pl.ds(b, _NSUB)], dsts[q],
                                     sem_a),
                    pltpu.async_copy(typ_h.at[pl.ds(b * _SUB, _CH)],
                                     typs[q], sem_a)]

        def wait_led(q):
            pltpu.make_async_copy(dst2_h.at[pl.ds(0, _NSUB)], dsts[q],
                                  sem_a).wait()
            pltpu.make_async_copy(typ_h.at[pl.ds(0, _CH)], typs[q],
                                  sem_a).wait()

        def cpair(q):
            for j in range(_NSUB):
                for g in range(_SUB // 16):
                    sl = pl.ds(g * 16, 16)
                    pairs[q][j, sl] = (dsts[q][j, sl] * R
                                       + typs[q][pl.ds(j * _SUB + g * 16,
                                                       16)])

        def fire_sc(q, vals):
            return [pltpu.async_copy(vals, cnt_sp.at[pairs[q].at[j]],
                                     sem_b, add=True)
                    for j in range(_NSUB)]

        def drain_sc(q):
            for j in range(_NSUB):
                pltpu.make_async_copy(onesv, cnt_sp.at[pairs[q].at[j]],
                                      sem_b).wait()

        blk0 = s * (LPT // _SUB)
        fire_sc(0, zb1.at[pl.ds(0, _SUB)])
        fire_sc(1, zb1.at[pl.ds(0, _SUB)])
        pltpu.sync_copy(dst2_h.at[pl.ds(blk0, _NSUB)], dst_a)
        pltpu.sync_copy(typ_h.at[pl.ds(blk0 * _SUB, _CH)], typ_a)

        def count_pair(t, carry):
            for q in range(2):
                i = 2 * t + q
                nx = led(jnp.minimum(i + 1, NCH - 1), 1 - q, blk0)
                drain_sc(q)
                cpair(q)
                fire_sc(q, onesv)
                for cp in nx:
                    cp.wait()
            return carry
        lax.fori_loop(0, NCH // 2, count_pair, 0)
        drain_sc(0)
        drain_sc(1)
        plsc.subcore_barrier()

        wblk0 = (c * _NT + s) * (LPW // _SUB)
        pltpu.sync_copy(dst2_h.at[pl.ds(wblk0, _NSUB)], dst_a)
        pltpu.sync_copy(typ_h.at[pl.ds(wblk0 * _SUB, _CH)], typ_a)

        def norm_pair(t, carry):
            for q in range(2):
                i = 2 * t + q
                nx = led(jnp.minimum(i + 1, NCHW - 1), 1 - q, wblk0)
                cpair(q)
                gcs = [pltpu.async_copy(cnt_sp.at[pairs[q].at[j]],
                                        cntv.at[pl.ds(j * _SUB, _SUB)],
                                        sem_b)
                       for j in range(_NSUB)]
                for cp in gcs:
                    cp.wait()
                for g in range(_CH // 16):
                    sl = pl.ds(g * 16, 16)
                    normv[sl] = 1.0 / cntv[sl]
                base = (wblk0 + i * _NSUB) * _SUB
                pltpu.sync_copy(normv, norm_h.at[pl.ds(base, _CH)])
                for cp in nx:
                    cp.wait()
            return carry
        lax.fori_loop(0, NCHW // 2, norm_pair, 0)

    scratch = [
        pltpu.VMEM_SHARED((CNTW,), jnp.float32),
        pltpu.VMEM((_NSUB, _SUB), jnp.int32),
        pltpu.VMEM((_NSUB, _SUB), jnp.int32),
        pltpu.VMEM((_CH,), jnp.int32),
        pltpu.VMEM((_CH,), jnp.int32),
        pltpu.VMEM((_NSUB, _SUB), jnp.int32),
        pltpu.VMEM((_NSUB, _SUB), jnp.int32),
        pltpu.VMEM((_CH,), jnp.float32),
        pltpu.VMEM((_CH,), jnp.float32),
        pltpu.VMEM((_SUB,), jnp.float32),
        pltpu.VMEM((4096,), jnp.float32),
        pltpu.SemaphoreType.DMA,
        pltpu.SemaphoreType.DMA,
    ]
    return body, scratch


def _build_layer(N, R, W, E_pad, layer1):
    LPT = E_pad // _NT if layer1 else E_pad // (_NC * _NT)
    NCH = LPT // _CH
    OPT8 = _ceil_to(-(-N // _NT), 8)
    NROW = _NT * OPT8
    LASTN = N - (_NT - 1) * OPT8

    def body(src_h, dst2_h, typ_h, norm_h, tab_h, out_h,
             acc_sp, src_a, src_b, typ_a, typ_b, dst_a, dst_b,
             idx_a, idx_b, nrm_a, nrm_b,
             rows_a, rows_b, rows2,
             sed_a, sed_b, sg_a, sg_b, ss):
        c = lax.axis_index("c")
        s = lax.axis_index("s")
        i16 = lax.broadcasted_iota(jnp.int32, (16,), 0)
        zf = jnp.zeros((16,), jnp.float32)
        srcs = (src_a, src_b)
        typs = (typ_a, typ_b)
        dsts = (dst_a, dst_b)
        idxs = (idx_a, idx_b)
        nrms = (nrm_a, nrm_b)
        rowss = (rows_a, rows_b)
        seds = (sed_a, sed_b)
        sgs = (sg_a, sg_b)

        def zrow(r, carry):
            for qq in range(W // 16):
                rows2[r, pl.ds(qq * 16, 16)] = zf
            return carry
        lax.fori_loop(0, _CH, zrow, 0)
        rz = s * OPT8
        nfull = OPT8 // _CH
        zcs = [pltpu.async_copy(rows2, acc_sp.at[pl.ds(rz + k * _CH, _CH)],
                                sg_a)
               for k in range(nfull)]
        rem = OPT8 - nfull * _CH
        if rem:
            zcs.append(pltpu.async_copy(
                rows2.at[pl.ds(0, rem)],
                acc_sp.at[pl.ds(rz + nfull * _CH, rem)], sg_a))
        for cp in zcs:
            cp.wait()
        plsc.subcore_barrier()

        if layer1:
            blk0 = s * (LPT // _SUB)
        else:
            blk0 = (c * _NT + s) * (LPT // _SUB)

        def blk(i):
            return blk0 + i * _NSUB

        def ebase(i):
            return blk(i) * _SUB

        def load_ed(i, q, sem):
            return [pltpu.async_copy(dst2_h.at[pl.ds(blk(i), _NSUB)],
                                     dsts[q], sem),
                    pltpu.async_copy(src_h.at[pl.ds(ebase(i), _CH)],
                                     srcs[q], sem),
                    pltpu.async_copy(typ_h.at[pl.ds(ebase(i), _CH)],
                                     typs[q], sem),
                    pltpu.async_copy(norm_h.at[pl.ds(ebase(i), _CH)],
                                     nrms[q], sem)]

        def wait_ed(q):
            pltpu.make_async_copy(dst2_h.at[pl.ds(0, _NSUB)], dsts[q],
                                  seds[q]).wait()
            pltpu.make_async_copy(src_h.at[pl.ds(0, _CH)], srcs[q],
                                  seds[q]).wait()
            pltpu.make_async_copy(typ_h.at[pl.ds(0, _CH)], typs[q],
                                  seds[q]).wait()
            pltpu.make_async_copy(norm_h.at[pl.ds(0, _CH)], nrms[q],
                                  seds[q]).wait()

        def comp_idx(q):
            for j in range(_NSUB):
                for g in range(_SUB // 16):
                    sl = pl.ds(g * 16, 16)
                    slf = pl.ds(j * _SUB + g * 16, 16)
                    sv = srcs[q][slf]
                    tv = typs[q][slf]
                    if layer1:
                        idxs[q][j, sl] = (tv * N + sv) * 2 + c
                    else:
                        idxs[q][j, sl] = sv * R + tv

        def fire_gather(q):
            return [pltpu.async_copy(tab_h.at[idxs[q].at[j]],
                                     rowss[q].at[pl.ds(j * _SUB, _SUB)],
                                     sgs[q])
                    for j in range(_NSUB)]

        def wait_gather(q):
            for j in range(_NSUB):
                pltpu.make_async_copy(tab_h.at[idxs[q].at[j]],
                                      rowss[q].at[pl.ds(j * _SUB, _SUB)],
                                      sgs[q]).wait()

        def scale(q):
            def sgroup(k, carry):
                fo = k * 16
                nv = nrms[q][pl.ds(fo, 16)]
                for h in range(2):
                    vals = []
                    for g2 in range(8):
                        e = fo + h * 8 + g2
                        s0 = nv[h * 8 + g2]
                        vals.append([rowss[q][e, pl.ds(c16 * 16, 16)] * s0
                                     for c16 in range(W // 16)])
                    for g2 in range(8):
                        e = fo + h * 8 + g2
                        for c16 in range(W // 16):
                            rows2[e, pl.ds(c16 * 16, 16)] = vals[g2][c16]
                return carry
            lax.fori_loop(0, _CH // 16, sgroup, 0)

        def fire_scatter(q):
            return [pltpu.async_copy(rows2.at[pl.ds(j * _SUB, _SUB)],
                                     acc_sp.at[dsts[q].at[j]], ss,
                                     add=True)
                    for j in range(_NSUB)]

        pltpu.sync_copy(dst2_h.at[pl.ds(blk(0), _NSUB)], dst_a)
        pltpu.sync_copy(src_h.at[pl.ds(ebase(0), _CH)], src_a)
        pltpu.sync_copy(typ_h.at[pl.ds(ebase(0), _CH)], typ_a)
        pltpu.sync_copy(norm_h.at[pl.ds(ebase(0), _CH)], nrm_a)
        comp_idx(0)
        fire_gather(0)
        load_ed(jnp.minimum(1, NCH - 1), 1, sed_b)

        def step(i, p):
            q, r = p, 1 - p
            wait_ed(r)
            comp_idx(r)
            wait_gather(q)
            fire_gather(r)
            scale(q)
            scs = fire_scatter(q)
            for cp in scs:
                cp.wait()
            load_ed(jnp.minimum(i + 2, NCH - 1), q, seds[q])

        def pair_step(t, carry):
            step(2 * t, 0)
            step(2 * t + 1, 1)
            return carry
        lax.fori_loop(0, NCH // 2, pair_step, 0)

        wait_gather(0)
        wait_ed(1)
        plsc.subcore_barrier()

        _write_stripes(acc_sp, out_h, c * N, s, OPT8, LASTN, ss)

    scratch = [
        pltpu.VMEM_SHARED((NROW, W), jnp.float32),
        pltpu.VMEM((_CH,), jnp.int32),
        pltpu.VMEM((_CH,), jnp.int32),
        pltpu.VMEM((_CH,), jnp.int32),
        pltpu.VMEM((_CH,), jnp.int32),
        pltpu.VMEM((_NSUB, _SUB), jnp.int32),
        pltpu.VMEM((_NSUB, _SUB), jnp.int32),
        pltpu.VMEM((_NSUB, _SUB), jnp.int32),
        pltpu.VMEM((_NSUB, _SUB), jnp.int32),
        pltpu.VMEM((_CH,), jnp.float32),
        pltpu.VMEM((_CH,), jnp.float32),
        pltpu.VMEM((_CH, W), jnp.float32),
        pltpu.VMEM((_CH, W), jnp.float32),
        pltpu.VMEM((_CH, W), jnp.float32),
        pltpu.SemaphoreType.DMA,
        pltpu.SemaphoreType.DMA,
        pltpu.SemaphoreType.DMA,
        pltpu.SemaphoreType.DMA,
        pltpu.SemaphoreType.DMA,
    ]
    return body, scratch


def _dense_body(h0, h1, r1, b1r, w2c, r2, xr_o, hr_o):
    h = jnp.concatenate([h0[...], h1[...]], axis=1)
    h = jnp.maximum(h + r1[...] + b1r[...], 0.0)
    xr_o[...] = jnp.dot(h, w2c[...], preferred_element_type=jnp.float32)
    hr_o[...] = jnp.dot(h, r2[...], preferred_element_type=jnp.float32)


def _finish_body(p0, p1, hr, b2r, o):
    x = p0[...] + p1[...] + hr[...] + b2r[...]
    o[...] = 1.0 / (1.0 + jnp.exp(-x))


def kernel(edge_index, edge_type, W1, root1, b1, W2, root2, b2):
    R, N, H = W1.shape
    L = W2.shape[2]
    E = edge_index.shape[1]
    Hh = H // 2

    E_pad = _ceil_to(E, 2 * _NC * _NT * _CH)
    pad = E_pad - E

    src = edge_index[0].astype(jnp.int32)
    dst = edge_index[1].astype(jnp.int32)
    typ = edge_type.astype(jnp.int32)
    src_p = jnp.concatenate([src, jnp.zeros((pad,), jnp.int32)])
    dst_p = jnp.concatenate([dst, jnp.full((pad,), N, jnp.int32)])
    typ_p = jnp.concatenate([typ, jnp.zeros((pad,), jnp.int32)])
    dst2 = dst_p.reshape(-1, _SUB)

    w1v = W1.reshape(R * N * 2, Hh)

    mesh = plsc.VectorSubcoreMesh(core_axis_name="c", subcore_axis_name="s")
    sc_params = pltpu.CompilerParams(needs_layout_passes=False,
                                     use_tc_tiling_on_sc=False)

    k0_body, k0_scratch = _build_k0(N, R, E_pad)
    norm_e = pl.kernel(
        k0_body,
        out_type=jax.ShapeDtypeStruct((E_pad,), jnp.float32),
        mesh=mesh,
        compiler_params=sc_params,
        scratch_types=k0_scratch,
    )(dst2, typ_p)

    k1_body, k1_scratch = _build_layer(N, R, Hh, E_pad, True)
    hpre = pl.kernel(
        k1_body,
        out_type=jax.ShapeDtypeStruct((_NC * N, Hh), jnp.float32),
        mesh=mesh,
        compiler_params=sc_params,
        scratch_types=k1_scratch,
    )(src_p, dst2, typ_p, norm_e, w1v)

    BN = 1000
    NB = N // BN
    w2c = jnp.transpose(W2, (1, 0, 2)).reshape(H, R * L)
    xr2d, hroot = pl.pallas_call(
        _dense_body,
        grid=(NB,),
        in_specs=[
            pl.BlockSpec((BN, Hh), lambda i: (i, 0)),
            pl.BlockSpec((BN, Hh), lambda i, _nb=NB: (i + _nb, 0)),
            pl.BlockSpec((BN, H), lambda i: (i, 0)),
            pl.BlockSpec((1, H), lambda i: (0, 0)),
            pl.BlockSpec((H, R * L), lambda i: (0, 0)),
            pl.BlockSpec((H, L), lambda i: (0, 0)),
        ],
        out_specs=[pl.BlockSpec((BN, R * L), lambda i: (i, 0)),
                   pl.BlockSpec((BN, L), lambda i: (i, 0))],
        out_shape=[jax.ShapeDtypeStruct((N, R * L), jnp.float32),
                   jax.ShapeDtypeStruct((N, L), jnp.float32)],
    )(hpre, hpre, root1, b1.reshape(1, H), w2c, root2)

    xrv = xr2d.reshape(N * R, L)

    k3_body, k3_scratch = _build_layer(N, R, L, E_pad, False)
    opart = pl.kernel(
        k3_body,
        out_type=jax.ShapeDtypeStruct((_NC * N, L), jnp.float32),
        mesh=mesh,
        compiler_params=sc_params,
        scratch_types=k3_scratch,
    )(src_p, dst2, typ_p, norm_e, xrv)

    out = pl.pallas_call(
        _finish_body,
        grid=(NB,),
        in_specs=[
            pl.BlockSpec((BN, L), lambda i: (i, 0)),
            pl.BlockSpec((BN, L), lambda i, _nb=NB: (i + _nb, 0)),
            pl.BlockSpec((BN, L), lambda i: (i, 0)),
            pl.BlockSpec((1, L), lambda i: (0, 0)),
        ],
        out_specs=pl.BlockSpec((BN, L), lambda i: (i, 0)),
        out_shape=jax.ShapeDtypeStruct((N, L), jnp.float32),
    )(opart, opart, hroot, b2.reshape(1, L))
    return out


# speedup vs baseline: 2.4153x; 1.0699x over previous
import jax
import jax.numpy as jnp
from jax import lax
from jax.experimental import pallas as pl
from jax.experimental.pallas import tpu as pltpu
from jax.experimental.pallas import tpu_sc as plsc

_CH = 256
_SUB = 128
_NSUB = _CH // _SUB
_NT = 16
_NC = 2


def _ceil_to(x, m):
    return -(-x // m) * m


def _write_stripes(acc_sp, out_h, row0, s, opt8, lastn, sem):
    kcom = lastn // _SUB
    ocs = []
    for k in range(kcom):
        ocs.append(pltpu.async_copy(
            acc_sp.at[pl.ds(s * opt8 + k * _SUB, _SUB)],
            out_h.at[pl.ds(row0 + s * opt8 + k * _SUB, _SUB)], sem))
    t_full = opt8 - kcom * _SUB
    t_last = lastn - kcom * _SUB
    if t_full == t_last:
        if t_full:
            ocs.append(pltpu.async_copy(
                acc_sp.at[pl.ds(s * opt8 + kcom * _SUB, t_full)],
                out_h.at[pl.ds(row0 + s * opt8 + kcom * _SUB, t_full)], sem))
    else:
        @pl.when(s < _NT - 1)
        def _():
            if t_full:
                pltpu.async_copy(
                    acc_sp.at[pl.ds(s * opt8 + kcom * _SUB, t_full)],
                    out_h.at[pl.ds(row0 + s * opt8 + kcom * _SUB, t_full)],
                    sem).wait()

        @pl.when(s == _NT - 1)
        def _():
            if t_last:
                pltpu.async_copy(
                    acc_sp.at[pl.ds(s * opt8 + kcom * _SUB, t_last)],
                    out_h.at[pl.ds(row0 + s * opt8 + kcom * _SUB, t_last)],
                    sem).wait()
    for cp in ocs:
        cp.wait()


def _build_k0(N, R, E_pad):
    LPT = E_pad // _NT
    NCH = LPT // _CH
    LPW = E_pad // (_NC * _NT)
    NCHW = LPW // _CH
    CWPT = _ceil_to(-(-(N * R + 1) // _NT), 128)
    CNTW = _NT * CWPT

    def body(dst2_h, typ_h, norm_h,
             cnt_sp, dst_a, dst_b, typ_a, typ_b, pair_a, pair_b,
             cntv, normv, onesv, zb1, sem_a, sem_b):
        c = lax.axis_index("c")
        s = lax.axis_index("s")
        on = jnp.ones((16,), jnp.float32)
        zf = jnp.zeros((16,), jnp.float32)
        zi = jnp.zeros((16,), jnp.int32)
        dsts = (dst_a, dst_b)
        typs = (typ_a, typ_b)
        pairs = (pair_a, pair_b)

        for g in range(_SUB // 16):
            onesv[pl.ds(g * 16, 16)] = on
        for q in range(2):
            for j in range(_NSUB):
                for g in range(_SUB // 16):
                    pairs[q][j, pl.ds(g * 16, 16)] = zi

        def zrow1(k, carry):
            zb1[pl.ds(k * 16, 16)] = zf
            return carry
        lax.fori_loop(0, 4096 // 16, zrow1, 0)

        wz = s * CWPT
        wfull = CWPT // 4096
        zcs = []
        for k in range(wfull):
            zcs.append(pltpu.async_copy(
                zb1, cnt_sp.at[pl.ds(wz + k * 4096, 4096)], sem_a))
        wrem = CWPT - wfull * 4096
        if wrem:
            zcs.append(pltpu.async_copy(
                zb1.at[pl.ds(0, wrem)],
                cnt_sp.at[pl.ds(wz + wfull * 4096, wrem)], sem_a))
        for cp in zcs:
            cp.wait()
        plsc.subcore_barrier()

        def led(i, q, blk_base):
            b = blk_base + i * _NSUB
            return [pltpu.async_copy(dst2_h.at[pl.ds(b, _NSUB)], dsts[q],
                                     sem_a),
                    pltpu.async_copy(typ_h.at[pl.ds(b * _SUB, _CH)],
                                     typs[q], sem_a)]

        def wait_led(q):
            pltpu.make_async_copy(dst2_h.at[pl.ds(0, _NSUB)], dsts[q],
                                  sem_a).wait()
            pltpu.make_async_copy(typ_h.at[pl.ds(0, _CH)], typs[q],
                                  sem_a).wait()

        def cpair(q):
            for j in range(_NSUB):
                for g in range(_SUB // 16):
                    sl = pl.ds(g * 16, 16)
                    pairs[q][j, sl] = (dsts[q][j, sl] * R
                                       + typs[q][pl.ds(j * _SUB + g * 16,
                                                       16)])

        def fire_sc(q, vals):
            return [pltpu.async_copy(vals, cnt_sp.at[pairs[q].at[j]],
                                     sem_b, add=True)
                    for j in range(_NSUB)]

        def drain_sc(q):
            for j in range(_NSUB):
                pltpu.make_async_copy(onesv, cnt_sp.at[pairs[q].at[j]],
                                      sem_b).wait()

        blk0 = s * (LPT // _SUB)
        fire_sc(0, zb1.at[pl.ds(0, _SUB)])
        fire_sc(1, zb1.at[pl.ds(0, _SUB)])
        pltpu.sync_copy(dst2_h.at[pl.ds(blk0, _NSUB)], dst_a)
        pltpu.sync_copy(typ_h.at[pl.ds(blk0 * _SUB, _CH)], typ_a)

        def count_pair(t, carry):
            for q in range(2):
                i = 2 * t + q
                nx = led(jnp.minimum(i + 1, NCH - 1), 1 - q, blk0)
                drain_sc(q)
                cpair(q)
                fire_sc(q, onesv)
                for cp in nx:
                    cp.wait()
            return carry
        lax.fori_loop(0, NCH // 2, count_pair, 0)
        drain_sc(0)
        drain_sc(1)
        plsc.subcore_barrier()

        wblk0 = (c * _NT + s) * (LPW // _SUB)
        pltpu.sync_copy(dst2_h.at[pl.ds(wblk0, _NSUB)], dst_a)
        pltpu.sync_copy(typ_h.at[pl.ds(wblk0 * _SUB, _CH)], typ_a)

        def norm_pair(t, carry):
            for q in range(2):
                i = 2 * t + q
                nx = led(jnp.minimum(i + 1, NCHW - 1), 1 - q, wblk0)
                cpair(q)
                gcs = [pltpu.async_copy(cnt_sp.at[pairs[q].at[j]],
                                        cntv.at[pl.ds(j * _SUB, _SUB)],
                                        sem_b)
                       for j in range(_NSUB)]
                for cp in gcs:
                    cp.wait()
                for g in range(_CH // 16):
                    sl = pl.ds(g * 16, 16)
                    normv[sl] = 1.0 / cntv[sl]
                base = (wblk0 + i * _NSUB) * _SUB
                pltpu.sync_copy(normv, norm_h.at[pl.ds(base, _CH)])
                for cp in nx:
                    cp.wait()
            return carry
        lax.fori_loop(0, NCHW // 2, norm_pair, 0)

    scratch = [
        pltpu.VMEM_SHARED((CNTW,), jnp.float32),
        pltpu.VMEM((_NSUB, _SUB), jnp.int32),
        pltpu.VMEM((_NSUB, _SUB), jnp.int32),
        pltpu.VMEM((_CH,), jnp.int32),
        pltpu.VMEM((_CH,), jnp.int32),
        pltpu.VMEM((_NSUB, _SUB), jnp.int32),
        pltpu.VMEM((_NSUB, _SUB), jnp.int32),
        pltpu.VMEM((_CH,), jnp.float32),
        pltpu.VMEM((_CH,), jnp.float32),
        pltpu.VMEM((_SUB,), jnp.float32),
        pltpu.VMEM((4096,), jnp.float32),
        pltpu.SemaphoreType.DMA,
        pltpu.SemaphoreType.DMA,
    ]
    return body, scratch


def _build_layer(N, R, W, E_pad, layer1):
    LPT = E_pad // _NT if layer1 else E_pad // (_NC * _NT)
    NCH = LPT // _CH
    OPT8 = _ceil_to(-(-N // _NT), 8)
    NROW = _NT * OPT8
    LASTN = N - (_NT - 1) * OPT8

    def body(src_h, dst2_h, typ_h, norm_h, tab_h, out_h,
             acc_sp, src_a, src_b, typ_a, typ_b, dst_a, dst_b,
             dsc_a, dsc_b, idx_a, idx_b, nrm_a, nrm_b,
             rows_a, rows_b,
             sed_a, sed_b, sg_a, sg_b, ss):
        c = lax.axis_index("c")
        s = lax.axis_index("s")
        i16 = lax.broadcasted_iota(jnp.int32, (16,), 0)
        zf = jnp.zeros((16,), jnp.float32)
        srcs = (src_a, src_b)
        typs = (typ_a, typ_b)
        dsts = (dst_a, dst_b)
        idxs = (idx_a, idx_b)
        nrms = (nrm_a, nrm_b)
        rowss = (rows_a, rows_b)
        dscs = (dsc_a, dsc_b)
        seds = (sed_a, sed_b)
        sgs = (sg_a, sg_b)

        def zrow(r, carry):
            for qq in range(W // 16):
                rows_a[r, pl.ds(qq * 16, 16)] = zf
            return carry
        lax.fori_loop(0, _CH, zrow, 0)
        rz = s * OPT8
        nfull = OPT8 // _CH
        zcs = [pltpu.async_copy(rows_a, acc_sp.at[pl.ds(rz + k * _CH, _CH)],
                                sg_a)
               for k in range(nfull)]
        rem = OPT8 - nfull * _CH
        if rem:
            zcs.append(pltpu.async_copy(
                rows_a.at[pl.ds(0, rem)],
                acc_sp.at[pl.ds(rz + nfull * _CH, rem)], sg_a))
        for cp in zcs:
            cp.wait()
        plsc.subcore_barrier()

        if layer1:
            blk0 = s * (LPT // _SUB)
        else:
            blk0 = (c * _NT + s) * (LPT // _SUB)

        def blk(i):
            return blk0 + i * _NSUB

        def ebase(i):
            return blk(i) * _SUB

        def load_ed(i, q, sem):
            return [pltpu.async_copy(dst2_h.at[pl.ds(blk(i), _NSUB)],
                                     dsts[q], sem),
                    pltpu.async_copy(src_h.at[pl.ds(ebase(i), _CH)],
                                     srcs[q], sem),
                    pltpu.async_copy(typ_h.at[pl.ds(ebase(i), _CH)],
                                     typs[q], sem),
                    pltpu.async_copy(norm_h.at[pl.ds(ebase(i), _CH)],
                                     nrms[q], sem)]

        def wait_ed(q):
            pltpu.make_async_copy(dst2_h.at[pl.ds(0, _NSUB)], dsts[q],
                                  seds[q]).wait()
            pltpu.make_async_copy(src_h.at[pl.ds(0, _CH)], srcs[q],
                                  seds[q]).wait()
            pltpu.make_async_copy(typ_h.at[pl.ds(0, _CH)], typs[q],
                                  seds[q]).wait()
            pltpu.make_async_copy(norm_h.at[pl.ds(0, _CH)], nrms[q],
                                  seds[q]).wait()

        def comp_idx(q):
            for j in range(_NSUB):
                for g in range(_SUB // 16):
                    sl = pl.ds(g * 16, 16)
                    slf = pl.ds(j * _SUB + g * 16, 16)
                    sv = srcs[q][slf]
                    tv = typs[q][slf]
                    if layer1:
                        idxs[q][j, sl] = (tv * N + sv) * 2 + c
                    else:
                        idxs[q][j, sl] = sv * R + tv

        def fire_gather(q):
            return [pltpu.async_copy(tab_h.at[idxs[q].at[j]],
                                     rowss[q].at[pl.ds(j * _SUB, _SUB)],
                                     sgs[q])
                    for j in range(_NSUB)]

        def wait_gather(q):
            for j in range(_NSUB):
                pltpu.make_async_copy(tab_h.at[idxs[q].at[j]],
                                      rowss[q].at[pl.ds(j * _SUB, _SUB)],
                                      sgs[q]).wait()

        def scale(q):
            def sgroup(k, carry):
                fo = k * 16
                nv = nrms[q][pl.ds(fo, 16)]
                for h in range(2):
                    vals = []
                    for g2 in range(8):
                        e = fo + h * 8 + g2
                        s0 = nv[h * 8 + g2]
                        vals.append([rowss[q][e, pl.ds(c16 * 16, 16)] * s0
                                     for c16 in range(W // 16)])
                    for g2 in range(8):
                        e = fo + h * 8 + g2
                        for c16 in range(W // 16):
                            rowss[q][e, pl.ds(c16 * 16, 16)] = vals[g2][c16]
                return carry
            lax.fori_loop(0, _CH // 16, sgroup, 0)

        def fire_scatter(q):
            for j in range(_NSUB):
                for g in range(_SUB // 16):
                    sl = pl.ds(g * 16, 16)
                    dscs[q][j, sl] = dsts[q][j, sl]
            return [pltpu.async_copy(rowss[q].at[pl.ds(j * _SUB, _SUB)],
                                     acc_sp.at[dscs[q].at[j]], ss,
                                     add=True)
                    for j in range(_NSUB)]

        def drain_scatter(q):
            for j in range(_NSUB):
                pltpu.make_async_copy(rowss[q].at[pl.ds(j * _SUB, _SUB)],
                                      acc_sp.at[dscs[q].at[j]], ss).wait()

        pltpu.sync_copy(dst2_h.at[pl.ds(blk(0), _NSUB)], dst_a)
        pltpu.sync_copy(src_h.at[pl.ds(ebase(0), _CH)], src_a)
        pltpu.sync_copy(typ_h.at[pl.ds(ebase(0), _CH)], typ_a)
        pltpu.sync_copy(norm_h.at[pl.ds(ebase(0), _CH)], nrm_a)
        comp_idx(0)
        fire_gather(0)
        load_ed(jnp.minimum(1, NCH - 1), 1, sed_b)

        def step(i, p):
            q, r = p, 1 - p
            wait_ed(r)
            comp_idx(r)
            wait_gather(q)

            @pl.when(i >= 1)
            def _():
                drain_scatter(r)
            fire_gather(r)
            scale(q)
            fire_scatter(q)
            load_ed(jnp.minimum(i + 2, NCH - 1), q, seds[q])

        def pair_step(t, carry):
            step(2 * t, 0)
            step(2 * t + 1, 1)
            return carry
        lax.fori_loop(0, NCH // 2, pair_step, 0)

        wait_gather(0)
        wait_ed(1)
        drain_scatter(1)
        plsc.subcore_barrier()

        _write_stripes(acc_sp, out_h, c * N, s, OPT8, LASTN, ss)

    scratch = [
        pltpu.VMEM_SHARED((NROW, W), jnp.float32),
        pltpu.VMEM((_CH,), jnp.int32),
        pltpu.VMEM((_CH,), jnp.int32),
        pltpu.VMEM((_CH,), jnp.int32),
        pltpu.VMEM((_CH,), jnp.int32),
        pltpu.VMEM((_NSUB, _SUB), jnp.int32),
        pltpu.VMEM((_NSUB, _SUB), jnp.int32),
        pltpu.VMEM((_NSUB, _SUB), jnp.int32),
        pltpu.VMEM((_NSUB, _SUB), jnp.int32),
        pltpu.VMEM((_NSUB, _SUB), jnp.int32),
        pltpu.VMEM((_NSUB, _SUB), jnp.int32),
        pltpu.VMEM((_CH,), jnp.float32),
        pltpu.VMEM((_CH,), jnp.float32),
        pltpu.VMEM((_CH, W), jnp.float32),
        pltpu.VMEM((_CH, W), jnp.float32),
        pltpu.SemaphoreType.DMA,
        pltpu.SemaphoreType.DMA,
        pltpu.SemaphoreType.DMA,
        pltpu.SemaphoreType.DMA,
        pltpu.SemaphoreType.DMA,
    ]
    return body, scratch


def _dense_body(h0, h1, r1, b1r, w2c, r2, xr_o, hr_o):
    h = jnp.concatenate([h0[...], h1[...]], axis=1)
    h = jnp.maximum(h + r1[...] + b1r[...], 0.0)
    xr_o[...] = jnp.dot(h, w2c[...], preferred_element_type=jnp.float32)
    hr_o[...] = jnp.dot(h, r2[...], preferred_element_type=jnp.float32)


def _finish_body(p0, p1, hr, b2r, o):
    x = p0[...] + p1[...] + hr[...] + b2r[...]
    o[...] = 1.0 / (1.0 + jnp.exp(-x))


def kernel(edge_index, edge_type, W1, root1, b1, W2, root2, b2):
    R, N, H = W1.shape
    L = W2.shape[2]
    E = edge_index.shape[1]
    Hh = H // 2

    E_pad = _ceil_to(E, 2 * _NC * _NT * _CH)
    pad = E_pad - E

    src = edge_index[0].astype(jnp.int32)
    dst = edge_index[1].astype(jnp.int32)
    typ = edge_type.astype(jnp.int32)
    src_p = jnp.concatenate([src, jnp.zeros((pad,), jnp.int32)])
    dst_p = jnp.concatenate([dst, jnp.full((pad,), N, jnp.int32)])
    typ_p = jnp.concatenate([typ, jnp.zeros((pad,), jnp.int32)])
    dst2 = dst_p.reshape(-1, _SUB)

    w1v = W1.reshape(R * N * 2, Hh)

    mesh = plsc.VectorSubcoreMesh(core_axis_name="c", subcore_axis_name="s")
    sc_params = pltpu.CompilerParams(needs_layout_passes=False,
                                     use_tc_tiling_on_sc=False)

    k0_body, k0_scratch = _build_k0(N, R, E_pad)
    norm_e = pl.kernel(
        k0_body,
        out_type=jax.ShapeDtypeStruct((E_pad,), jnp.float32),
        mesh=mesh,
        compiler_params=sc_params,
        scratch_types=k0_scratch,
    )(dst2, typ_p)

    k1_body, k1_scratch = _build_layer(N, R, Hh, E_pad, True)
    hpre = pl.kernel(
        k1_body,
        out_type=jax.ShapeDtypeStruct((_NC * N, Hh), jnp.float32),
        mesh=mesh,
        compiler_params=sc_params,
        scratch_types=k1_scratch,
    )(src_p, dst2, typ_p, norm_e, w1v)

    BN = 1000
    NB = N // BN
    w2c = jnp.transpose(W2, (1, 0, 2)).reshape(H, R * L)
    xr2d, hroot = pl.pallas_call(
        _dense_body,
        grid=(NB,),
        in_specs=[
            pl.BlockSpec((BN, Hh), lambda i: (i, 0)),
            pl.BlockSpec((BN, Hh), lambda i, _nb=NB: (i + _nb, 0)),
            pl.BlockSpec((BN, H), lambda i: (i, 0)),
            pl.BlockSpec((1, H), lambda i: (0, 0)),
            pl.BlockSpec((H, R * L), lambda i: (0, 0)),
            pl.BlockSpec((H, L), lambda i: (0, 0)),
        ],
        out_specs=[pl.BlockSpec((BN, R * L), lambda i: (i, 0)),
                   pl.BlockSpec((BN, L), lambda i: (i, 0))],
        out_shape=[jax.ShapeDtypeStruct((N, R * L), jnp.float32),
                   jax.ShapeDtypeStruct((N, L), jnp.float32)],
    )(hpre, hpre, root1, b1.reshape(1, H), w2c, root2)

    xrv = xr2d.reshape(N * R, L)

    k3_body, k3_scratch = _build_layer(N, R, L, E_pad, False)
    opart = pl.kernel(
        k3_body,
        out_type=jax.ShapeDtypeStruct((_NC * N, L), jnp.float32),
        mesh=mesh,
        compiler_params=sc_params,
        scratch_types=k3_scratch,
    )(src_p, dst2, typ_p, norm_e, xrv)

    out = pl.pallas_call(
        _finish_body,
        grid=(NB,),
        in_specs=[
            pl.BlockSpec((BN, L), lambda i: (i, 0)),
            pl.BlockSpec((BN, L), lambda i, _nb=NB: (i + _nb, 0)),
            pl.BlockSpec((BN, L), lambda i: (i, 0)),
            pl.BlockSpec((1, L), lambda i: (0, 0)),
        ],
        out_specs=pl.BlockSpec((BN, L), lambda i: (i, 0)),
        out_shape=jax.ShapeDtypeStruct((N, L), jnp.float32),
    )(opart, opart, hroot, b2.reshape(1, L))
    return out
